# Initial kernel scaffold; baseline (speedup 1.0000x reference)
import jax
import jax.numpy as jnp
from jax.experimental import pallas as pl


def kernel(x_atoms, x_frags, edge_attr, node_features_bonds, edge_attr_bonds, params, edge_index, frag_index, batch, frag_batch, atom_to_frag_ids, edge_index_bonds_graph):
    raise NotImplementedError("write your pallas kernel here")


# speedup vs baseline: 8.3689x; 8.3689x over previous
import functools

import jax
import jax.numpy as jnp
from jax import lax
from jax.experimental import pallas as pl
from jax.experimental.pallas import tpu as pltpu
from jax.experimental.pallas import tpu_sc as plsc

N = 10000
E = 160000
NF = 2000
EF = 4000
NB = 160000
BB = 256
EMB = 128
NC = 2
NS = 16
NW = NC * NS

f32 = jnp.float32
i32 = jnp.int32


def _mesh():
    return plsc.VectorSubcoreMesh(
        core_axis_name="c", subcore_axis_name="s", num_cores=NC, num_subcores=NS)


def _wid():
    return lax.axis_index("s") * NC + lax.axis_index("c")


def _row_split(total):
    main = (total // (8 * NS)) * 8
    extra_tiles = (total - main * NS) // 8
    extra_base = main * NS
    return main, extra_tiles, extra_base


def _zero_accum(z_h, accum, s, total):
    main, extra_tiles, extra_base = _row_split(total)
    base = pl.multiple_of(s * main, 8)
    pltpu.sync_copy(z_h.at[pl.ds(0, main)], accum.at[pl.ds(base, main)])
    if extra_tiles:
        @pl.when(s < extra_tiles)
        def _():
            eb = pl.multiple_of(extra_base + s * 8, 8)
            pltpu.sync_copy(z_h.at[pl.ds(0, 8)], accum.at[pl.ds(eb, 8)])


def _writeback(accum, out0, out1, s, total):
    c = lax.axis_index("c")
    main, extra_tiles, extra_base = _row_split(total)

    def _copy_ranges(out):
        base = pl.multiple_of(s * main, 8)
        pltpu.sync_copy(accum.at[pl.ds(base, main)], out.at[pl.ds(base, main)])
        if extra_tiles:
            @pl.when(s < extra_tiles)
            def _():
                eb = pl.multiple_of(extra_base + s * 8, 8)
                pltpu.sync_copy(accum.at[pl.ds(eb, 8)], out.at[pl.ds(eb, 8)])

    @pl.when(c == 0)
    def _():
        _copy_ranges(out0)

    @pl.when(c == 1)
    def _():
        _copy_ranges(out1)


def _u_build(eib0, eib1, ei1, eab, nfb_pad, zrows, z1d):
    nchunks = E // 128
    per = -(-nchunks // NW)
    onedim = jax.ShapeDtypeStruct((N,), f32)

    @functools.partial(
        pl.kernel,
        out_type=(jax.ShapeDtypeStruct((N, 16), f32),
                  jax.ShapeDtypeStruct((N, 16), f32),
                  onedim, onedim, onedim, onedim, onedim, onedim),
        mesh=_mesh(),
        scratch_types=[
            pltpu.VMEM_SHARED((N, 16), f32),
            pltpu.VMEM_SHARED((N,), f32),
            pltpu.VMEM_SHARED((N,), f32),
            pltpu.VMEM_SHARED((N,), f32),
            pltpu.VMEM((128,), i32),
            pltpu.VMEM((128,), i32),
            pltpu.VMEM((128,), i32),
            pltpu.VMEM((128,), f32),
            pltpu.VMEM((128,), f32),
            pltpu.VMEM((128,), f32),
            pltpu.VMEM((128, 16), f32),
            pltpu.VMEM((632,), f32),
            pltpu.SemaphoreType.DMA,
        ],
        compiler_params=pltpu.CompilerParams(use_tc_tiling_on_sc=False),
    )
    def k(eib0_h, eib1_h, ei1_h, eab_h, nfb_h, z_h, z1_h,
          out0, out1, t1a, t1b, tda, tdb, dga, dgb,
          accum, acc1, accd, accg, b0, b1, dsta, eb, ones_b, half_b, rows,
          bounce, sem):
        s = lax.axis_index("s")
        w = _wid()
        _zero_accum(z_h, accum, s, N)
        main, extra_tiles, extra_base = _row_split(N)
        base1 = pl.multiple_of(s * main, 8)
        pltpu.sync_copy(z1_h.at[pl.ds(0, main)], bounce.at[pl.ds(0, main)])
        for acc in (acc1, accd, accg):
            pltpu.sync_copy(bounce.at[pl.ds(0, main)], acc.at[pl.ds(base1, main)])
        if extra_tiles:
            @pl.when(s < extra_tiles)
            def _():
                eb_ = pl.multiple_of(extra_base + s * 8, 8)
                for acc in (acc1, accd, accg):
                    pltpu.sync_copy(bounce.at[pl.ds(0, 8)], acc.at[pl.ds(eb_, 8)])
        for j in range(8):
            ones_b[pl.ds(j * 16, 16)] = jnp.ones((16,), f32)
            half_b[pl.ds(j * 16, 16)] = jnp.full((16,), 1.5, f32)
        plsc.subcore_barrier()

        def pass_a(ci, carry):
            idx = w + ci * NW

            @pl.when(idx < nchunks)
            def _():
                off = pl.multiple_of(idx * 128, 128)
                pltpu.sync_copy(eib0_h.at[pl.ds(off, 128)], b0)
                pltpu.async_copy(ei1_h.at[b0], dsta, sem).wait()
                pltpu.sync_copy(eib1_h.at[pl.ds(off, 128)], b1)
                pltpu.async_copy(nfb_h.at[b1], rows, sem).wait()
                pltpu.sync_copy(eab_h.at[pl.ds(off, 128)], eb)
                pltpu.sync_copy(rows, accum.at[dsta], add=True)
                pltpu.sync_copy(eb, acc1.at[dsta], add=True)
                pltpu.sync_copy(ones_b, accd.at[dsta], add=True)
            return carry

        lax.fori_loop(0, per, pass_a, 0)

        def pass_b(ci, carry):
            idx = w + ci * NW

            @pl.when(idx < nchunks)
            def _():
                off = pl.multiple_of(idx * 128, 128)
                pltpu.sync_copy(nfb_h.at[pl.ds(off, 128)], rows)
                pltpu.sync_copy(ei1_h.at[pl.ds(off, 128)], b0)
                pltpu.sync_copy(rows, accum.at[b0], add=True)
                pltpu.sync_copy(half_b, acc1.at[b0], add=True)
                pltpu.sync_copy(ones_b, accd.at[b0], add=True)
                pltpu.sync_copy(ones_b, accg.at[b0], add=True)
            return carry

        lax.fori_loop(0, per, pass_b, 0)
        plsc.subcore_barrier()
        _writeback(accum, out0, out1, s, N)
        c = lax.axis_index("c")

        def wb1(acc, oa, ob):
            nb_ = main + (8 if extra_tiles else 0)
            pltpu.sync_copy(acc.at[pl.ds(base1, main)], bounce.at[pl.ds(0, main)])
            if extra_tiles:
                @pl.when(s < extra_tiles)
                def _():
                    eb_ = pl.multiple_of(extra_base + s * 8, 8)
                    pltpu.sync_copy(acc.at[pl.ds(eb_, 8)],
                                    bounce.at[pl.ds(main, 8)])

            @pl.when(c == 0)
            def _():
                pltpu.sync_copy(bounce.at[pl.ds(0, main)],
                                oa.at[pl.ds(base1, main)])

            @pl.when(c == 1)
            def _():
                pltpu.sync_copy(bounce.at[pl.ds(0, main)],
                                ob.at[pl.ds(base1, main)])
            if extra_tiles:
                @pl.when((s < extra_tiles) & (c == 0))
                def _():
                    eb_ = pl.multiple_of(extra_base + s * 8, 8)
                    pltpu.sync_copy(bounce.at[pl.ds(main, 8)],
                                    oa.at[pl.ds(eb_, 8)])

                @pl.when((s < extra_tiles) & (c == 1))
                def _():
                    eb_ = pl.multiple_of(extra_base + s * 8, 8)
                    pltpu.sync_copy(bounce.at[pl.ds(main, 8)],
                                    ob.at[pl.ds(eb_, 8)])

        wb1(acc1, t1a, t1b)
        wb1(accd, tda, tdb)
        wb1(accg, dga, dgb)

    return k(eib0, eib1, ei1, eab, nfb_pad, zrows, z1d)


def _spmv(x, ei0, ei1, zrows):
    nchunks = E // 128
    per = -(-nchunks // NW)

    @functools.partial(
        pl.kernel,
        out_type=(jax.ShapeDtypeStruct((N, EMB), f32),
                  jax.ShapeDtypeStruct((N, EMB), f32)),
        mesh=_mesh(),
        scratch_types=[
            pltpu.VMEM_SHARED((N, EMB), f32),
            pltpu.VMEM((128,), i32),
            pltpu.VMEM((128,), i32),
            pltpu.VMEM((128, EMB), f32),
            pltpu.SemaphoreType.DMA,
        ],
    )
    def k(x_h, ei0_h, ei1_h, z_h, out0, out1, accum, srcb, dstb, rows, sem):
        s = lax.axis_index("s")
        w = _wid()
        _zero_accum(z_h, accum, s, N)
        plsc.subcore_barrier()

        def body(ci, carry):
            idx = w + ci * NW

            @pl.when(idx < nchunks)
            def _():
                off = pl.multiple_of(idx * 128, 128)
                pltpu.sync_copy(ei0_h.at[pl.ds(off, 128)], srcb)
                pltpu.async_copy(x_h.at[srcb], rows, sem).wait()
                pltpu.sync_copy(ei1_h.at[pl.ds(off, 128)], dstb)
                pltpu.sync_copy(rows, accum.at[dstb], add=True)
            return carry

        lax.fori_loop(0, per, body, 0)
        plsc.subcore_barrier()
        _writeback(accum, out0, out1, s, N)

    return k(x, ei0, ei1, zrows)


def _frag_pool(pre3, a2f, batch, z128):
    C = 80
    nchunks = N // C
    per = -(-nchunks // NW)

    @functools.partial(
        pl.kernel,
        out_type=(jax.ShapeDtypeStruct((NF, EMB), f32),
                  jax.ShapeDtypeStruct((NF, EMB), f32),
                  jax.ShapeDtypeStruct((BB, EMB), f32),
                  jax.ShapeDtypeStruct((BB, EMB), f32)),
        mesh=_mesh(),
        scratch_types=[
            pltpu.VMEM_SHARED((NF, EMB), f32),
            pltpu.VMEM_SHARED((BB, EMB), f32),
            pltpu.VMEM((C,), i32),
            pltpu.VMEM((C,), i32),
            pltpu.VMEM((C, EMB), f32),
        ],
    )
    def k(x_h, a2f_h, batch_h, z_h, xf0, xf1, xap0, xap1,
          xfacc, xapacc, dstf, dstb, rows):
        s = lax.axis_index("s")
        w = _wid()
        _zero_accum(z_h, xfacc, s, NF)
        _zero_accum(z_h, xapacc, s, BB)
        plsc.subcore_barrier()

        def body(ci, carry):
            idx = w + ci * NW

            @pl.when(idx < nchunks)
            def _():
                off = pl.multiple_of(idx * C, 8)
                pltpu.sync_copy(x_h.at[pl.ds(off, C)], rows)
                pltpu.sync_copy(a2f_h.at[pl.ds(off, C)], dstf)
                pltpu.sync_copy(rows, xfacc.at[dstf], add=True)

                def relu_row(r, cc):
                    for j in range(8):
                        v = rows[r, pl.ds(j * 16, 16)]
                        rows[r, pl.ds(j * 16, 16)] = jnp.maximum(v, 0.0)
                    return cc

                lax.fori_loop(0, C, relu_row, 0)
                pltpu.sync_copy(batch_h.at[pl.ds(off, C)], dstb)
                pltpu.sync_copy(rows, xapacc.at[dstb], add=True)
            return carry

        lax.fori_loop(0, per, body, 0)
        plsc.subcore_barrier()
        _writeback(xfacc, xf0, xf1, s, NF)
        _writeback(xapacc, xap0, xap1, s, BB)

    return k(pre3, a2f, batch, z128)


def _frag_gather(xf0, xf1, fs, ft, z128):
    C = 40
    nchunks = EF // C
    per = -(-nchunks // NW)

    @functools.partial(
        pl.kernel,
        out_type=(jax.ShapeDtypeStruct((NF, EMB), f32),
                  jax.ShapeDtypeStruct((NF, EMB), f32)),
        mesh=_mesh(),
        scratch_types=[
            pltpu.VMEM_SHARED((NF, EMB), f32),
            pltpu.VMEM((C,), i32),
            pltpu.VMEM((C,), i32),
            pltpu.VMEM((C, EMB), f32),
            pltpu.VMEM((C, EMB), f32),
            pltpu.SemaphoreType.DMA,
        ],
    )
    def k(xf0_h, xf1_h, fs_h, ft_h, z_h, out0, out1,
          accum, ib, db, r0, r1, sem):
        s = lax.axis_index("s")
        w = _wid()
        _zero_accum(z_h, accum, s, NF)
        plsc.subcore_barrier()

        def body(ci, carry):
            idx = w + ci * NW

            @pl.when(idx < nchunks)
            def _():
                off = pl.multiple_of(idx * C, 8)
                pltpu.sync_copy(fs_h.at[pl.ds(off, C)], ib)
                pltpu.async_copy(xf0_h.at[ib], r0, sem).wait()
                pltpu.async_copy(xf1_h.at[ib], r1, sem).wait()

                def add_row(r, cc):
                    for j in range(8):
                        sl = pl.ds(j * 16, 16)
                        r0[r, sl] = r0[r, sl] + r1[r, sl]
                    return cc

                lax.fori_loop(0, C, add_row, 0)
                pltpu.sync_copy(ft_h.at[pl.ds(off, C)], db)
                pltpu.sync_copy(r0, accum.at[db], add=True)
            return carry

        lax.fori_loop(0, per, body, 0)
        plsc.subcore_barrier()
        _writeback(accum, out0, out1, s, NF)

    return k(xf0, xf1, fs, ft, z128)


def _layer_update(p0, p1, x, u0, u1, ue0, ue1, w, v, ve, b, relu):
    blk = 2000

    def body(p0_r, p1_r, x_r, u0_r, u1_r, ue0_r, ue1_r, w_r, v_r, ve_r, b_r,
             out_r):
        sx = p0_r[...] + p1_r[...] + x_r[...]
        acc = jnp.dot(sx, w_r[...], preferred_element_type=f32,
                      precision=lax.Precision.HIGHEST)
        acc = acc + jnp.dot(u0_r[...] + u1_r[...], v_r[...],
                            preferred_element_type=f32,
                      precision=lax.Precision.HIGHEST)
        acc = acc + jnp.dot(ue0_r[...] + ue1_r[...], ve_r[...],
                            preferred_element_type=f32,
                      precision=lax.Precision.HIGHEST)
        acc = acc + b_r[0:1, :]
        out_r[...] = jnp.maximum(acc, 0.0) if relu else acc

    return pl.pallas_call(
        body,
        grid=(N // blk,),
        in_specs=[
            pl.BlockSpec((blk, EMB), lambda i: (i, 0)),
            pl.BlockSpec((blk, EMB), lambda i: (i, 0)),
            pl.BlockSpec((blk, EMB), lambda i: (i, 0)),
            pl.BlockSpec((blk, 16), lambda i: (i, 0)),
            pl.BlockSpec((blk, 16), lambda i: (i, 0)),
            pl.BlockSpec((blk, 3), lambda i: (i, 0)),
            pl.BlockSpec((blk, 3), lambda i: (i, 0)),
            pl.BlockSpec((EMB, EMB), lambda i: (0, 0)),
            pl.BlockSpec((16, EMB), lambda i: (0, 0)),
            pl.BlockSpec((3, EMB), lambda i: (0, 0)),
            pl.BlockSpec((8, EMB), lambda i: (0, 0)),
        ],
        out_specs=pl.BlockSpec((blk, EMB), lambda i: (i, 0)),
        out_shape=jax.ShapeDtypeStruct((N, EMB), f32),
    )(p0, p1, x, u0, u1, ue0, ue1, w, v, ve, b)


def _tail(xap0, xap1, fs0, fs1, w1, b1, w2, b2, wa, wf, b3, wo, bo, oh):
    def body(xap0_r, xap1_r, fs0_r, fs1_r, w1_r, b1_r, w2_r, b2_r,
             wa_r, wf_r, b3_r, wo_r, bo_r, oh_r, out_r):
        fsum = fs0_r[...] + fs1_r[...]
        h = jnp.maximum(jnp.dot(fsum, w1_r[...], preferred_element_type=f32,
                      precision=lax.Precision.HIGHEST)
                        + b1_r[0:1, :], 0.0)
        xf = jnp.maximum(jnp.dot(h, w2_r[...], preferred_element_type=f32,
                      precision=lax.Precision.HIGHEST)
                         + b2_r[0:1, :], 0.0)
        xfp = lax.dot_general(oh_r[...], xf, (((0,), (0,)), ((), ())),
                              preferred_element_type=f32,
                              precision=lax.Precision.HIGHEST)
        xap = xap0_r[...] + xap1_r[...]
        hh = jnp.maximum(jnp.dot(xap, wa_r[...], preferred_element_type=f32,
                      precision=lax.Precision.HIGHEST)
                         + jnp.dot(xfp, wf_r[...], preferred_element_type=f32,
                      precision=lax.Precision.HIGHEST)
                         + b3_r[0:1, :], 0.0)
        out_r[...] = jnp.dot(hh, wo_r[...], preferred_element_type=f32,
                      precision=lax.Precision.HIGHEST) + bo_r[0:1, :]

    return pl.pallas_call(
        body,
        out_shape=jax.ShapeDtypeStruct((BB, EMB), f32),
    )(xap0, xap1, fs0, fs1, w1, b1, w2, b2, wa, wf, b3, wo, bo, oh)


def kernel(x_atoms, x_frags, edge_attr, node_features_bonds, edge_attr_bonds,
           params, edge_index, frag_index, batch, frag_batch, atom_to_frag_ids,
           edge_index_bonds_graph):
    ei0 = jnp.asarray(edge_index[0], i32)
    ei1 = jnp.asarray(edge_index[1], i32)
    eib0 = jnp.asarray(edge_index_bonds_graph[0], i32)
    eib1 = jnp.asarray(edge_index_bonds_graph[1], i32)
    fs = jnp.asarray(frag_index[0], i32)
    ft = jnp.asarray(frag_index[1], i32)
    a2f = jnp.asarray(atom_to_frag_ids, i32)
    bat = jnp.asarray(batch, i32)
    fbat = jnp.asarray(frag_batch, i32)
    eab = edge_attr_bonds[:, 0]

    nfb_pad = jnp.pad(node_features_bonds, ((0, 0), (0, 4)))
    x0_pad = jnp.pad(x_atoms, ((0, 0), (0, EMB - x_atoms.shape[1])))

    z625x16 = jnp.zeros((N // NS, 16), f32)
    z625x128 = jnp.zeros((N // NS, EMB), f32)
    z128x128 = jnp.zeros((128, EMB), f32)
    z1d = jnp.zeros((N // NS,), f32)

    u0, u1, t1a, t1b, tda, tdb, dga, dgb = _u_build(
        eib0, eib1, ei1, eab, nfb_pad, z625x16, z1d)
    ue0 = jnp.stack([t1a, tda, dga], axis=1)
    ue1 = jnp.stack([t1b, tdb, dgb], axis=1)

    layers = params["layers"]
    x = x0_pad
    pre3 = None
    for l in range(4):
        lp = layers[l]
        wl = lp["atom_embed"]["W"]
        if l == 0:
            wl = jnp.pad(wl, ((0, EMB - wl.shape[0]), (0, 0)))
        vl = jnp.concatenate([
            lp["edge_embed"]["W"],
            jnp.zeros((4, EMB), f32),
        ], axis=0)
        ve = jnp.concatenate([
            lp["edge_attr_bond_embed"]["W"],
            (lp["edge_attr_bond_embed"]["b"] + lp["edge_embed"]["b"])[None, :],
            lp["atom_embed"]["b"][None, :],
        ], axis=0)
        bl = jnp.broadcast_to(lp["atom_embed"]["b"][None, :], (8, EMB))
        p0, p1 = _spmv(x, ei0, ei1, z625x128)
        out = _layer_update(p0, p1, x, u0, u1, ue0, ue1, wl, vl, ve, bl,
                            relu=(l < 3))
        if l == 3:
            pre3 = out
        else:
            x = out

    xf0, xf1, xap0, xap1 = _frag_pool(pre3, a2f, bat, z128x128)
    fs0, fs1 = _frag_gather(xf0, xf1, fs, ft, z128x128)

    lp3 = layers[3]
    w1 = lp3["frag_mlp1"]["W"]
    b1 = jnp.broadcast_to(lp3["frag_mlp1"]["b"][None, :], (8, 2 * EMB))
    w2 = lp3["frag_mlp2"]["W"]
    b2 = jnp.broadcast_to(lp3["frag_mlp2"]["b"][None, :], (8, EMB))
    wa = params["lin1"]["W"][:EMB]
    wf = params["lin1"]["W"][EMB:]
    b3 = jnp.broadcast_to(params["lin1"]["b"][None, :], (8, 2 * EMB))
    wo = jnp.pad(params["out"]["W"], ((0, 0), (0, EMB - 1)))
    bo = jnp.pad(params["out"]["b"][None, :], ((0, 7), (0, EMB - 1)))
    oh = (fbat[:, None] == jnp.arange(BB)[None, :]).astype(f32)

    res = _tail(xap0, xap1, fs0, fs1, w1, b1, w2, b2, wa, wf, b3, wo, bo, oh)
    return res[:, :1]


# speedup vs baseline: 10.1229x; 1.2096x over previous
import functools

import jax
import jax.numpy as jnp
from jax import lax
from jax.experimental import pallas as pl
from jax.experimental.pallas import tpu as pltpu
from jax.experimental.pallas import tpu_sc as plsc

N = 10000
E = 160000
NF = 2000
EF = 4000
NB = 160000
BB = 256
EMB = 128
NC = 2
NS = 16
NW = NC * NS
EP = 163840

f32 = jnp.float32
i32 = jnp.int32


def _mesh():
    return plsc.VectorSubcoreMesh(
        core_axis_name="c", subcore_axis_name="s", num_cores=NC, num_subcores=NS)


def _wid():
    return lax.axis_index("s") * NC + lax.axis_index("c")


def _row_split(total):
    main = (total // (8 * NS)) * 8
    extra_tiles = (total - main * NS) // 8
    extra_base = main * NS
    return main, extra_tiles, extra_base


def _zero_accum(z_h, accum, s, total):
    main, extra_tiles, extra_base = _row_split(total)
    base = pl.multiple_of(s * main, 8)
    pltpu.sync_copy(z_h.at[pl.ds(0, main)], accum.at[pl.ds(base, main)])
    if extra_tiles:
        @pl.when(s < extra_tiles)
        def _():
            eb = pl.multiple_of(extra_base + s * 8, 8)
            pltpu.sync_copy(z_h.at[pl.ds(0, 8)], accum.at[pl.ds(eb, 8)])


def _writeback(accum, out0, out1, s, total):
    c = lax.axis_index("c")
    main, extra_tiles, extra_base = _row_split(total)

    def _copy_ranges(out):
        base = pl.multiple_of(s * main, 8)
        pltpu.sync_copy(accum.at[pl.ds(base, main)], out.at[pl.ds(base, main)])
        if extra_tiles:
            @pl.when(s < extra_tiles)
            def _():
                eb = pl.multiple_of(extra_base + s * 8, 8)
                pltpu.sync_copy(accum.at[pl.ds(eb, 8)], out.at[pl.ds(eb, 8)])

    @pl.when(c == 0)
    def _():
        _copy_ranges(out0)

    @pl.when(c == 1)
    def _():
        _copy_ranges(out1)


def _u_build(eib0, eib1, ei1, eab, nfb_pad, zrows, z1d):
    nchunks = E // 128
    per = -(-nchunks // NW)
    onedim = jax.ShapeDtypeStruct((N,), f32)

    @functools.partial(
        pl.kernel,
        out_type=(jax.ShapeDtypeStruct((N, 16), f32),
                  jax.ShapeDtypeStruct((N, 16), f32),
                  onedim, onedim, onedim, onedim, onedim, onedim),
        mesh=_mesh(),
        scratch_types=[
            pltpu.VMEM_SHARED((N, 16), f32),
            pltpu.VMEM_SHARED((N,), f32),
            pltpu.VMEM_SHARED((N,), f32),
            pltpu.VMEM_SHARED((N,), f32),
            pltpu.VMEM((128,), i32),
            pltpu.VMEM((128,), i32),
            pltpu.VMEM((128,), i32),
            pltpu.VMEM((128,), f32),
            pltpu.VMEM((128,), f32),
            pltpu.VMEM((128,), f32),
            pltpu.VMEM((128, 16), f32),
            pltpu.VMEM((632,), f32),
            pltpu.SemaphoreType.DMA,
        ],
        compiler_params=pltpu.CompilerParams(use_tc_tiling_on_sc=False),
    )
    def k(eib0_h, eib1_h, ei1_h, eab_h, nfb_h, z_h, z1_h,
          out0, out1, t1a, t1b, tda, tdb, dga, dgb,
          accum, acc1, accd, accg, b0, b1, dsta, eb, ones_b, half_b, rows,
          bounce, sem):
        s = lax.axis_index("s")
        w = _wid()
        _zero_accum(z_h, accum, s, N)
        main, extra_tiles, extra_base = _row_split(N)
        base1 = pl.multiple_of(s * main, 8)
        pltpu.sync_copy(z1_h.at[pl.ds(0, main)], bounce.at[pl.ds(0, main)])
        for acc in (acc1, accd, accg):
            pltpu.sync_copy(bounce.at[pl.ds(0, main)], acc.at[pl.ds(base1, main)])
        if extra_tiles:
            @pl.when(s < extra_tiles)
            def _():
                eb_ = pl.multiple_of(extra_base + s * 8, 8)
                for acc in (acc1, accd, accg):
                    pltpu.sync_copy(bounce.at[pl.ds(0, 8)], acc.at[pl.ds(eb_, 8)])
        for j in range(8):
            ones_b[pl.ds(j * 16, 16)] = jnp.ones((16,), f32)
            half_b[pl.ds(j * 16, 16)] = jnp.full((16,), 1.5, f32)
        plsc.subcore_barrier()

        def pass_a(ci, carry):
            idx = w + ci * NW

            @pl.when(idx < nchunks)
            def _():
                off = pl.multiple_of(idx * 128, 128)
                pltpu.sync_copy(eib0_h.at[pl.ds(off, 128)], b0)
                pltpu.async_copy(ei1_h.at[b0], dsta, sem).wait()
                pltpu.sync_copy(eib1_h.at[pl.ds(off, 128)], b1)
                pltpu.async_copy(nfb_h.at[b1], rows, sem).wait()
                pltpu.sync_copy(eab_h.at[pl.ds(off, 128)], eb)
                pltpu.sync_copy(rows, accum.at[dsta], add=True)
                pltpu.sync_copy(eb, acc1.at[dsta], add=True)
                pltpu.sync_copy(ones_b, accd.at[dsta], add=True)
            return carry

        lax.fori_loop(0, per, pass_a, 0)

        def pass_b(ci, carry):
            idx = w + ci * NW

            @pl.when(idx < nchunks)
            def _():
                off = pl.multiple_of(idx * 128, 128)
                pltpu.sync_copy(nfb_h.at[pl.ds(off, 128)], rows)
                pltpu.sync_copy(ei1_h.at[pl.ds(off, 128)], b0)
                pltpu.sync_copy(rows, accum.at[b0], add=True)
                pltpu.sync_copy(half_b, acc1.at[b0], add=True)
                pltpu.sync_copy(ones_b, accd.at[b0], add=True)
                pltpu.sync_copy(ones_b, accg.at[b0], add=True)
            return carry

        lax.fori_loop(0, per, pass_b, 0)
        plsc.subcore_barrier()
        _writeback(accum, out0, out1, s, N)
        c = lax.axis_index("c")

        def wb1(acc, oa, ob):
            nb_ = main + (8 if extra_tiles else 0)
            pltpu.sync_copy(acc.at[pl.ds(base1, main)], bounce.at[pl.ds(0, main)])
            if extra_tiles:
                @pl.when(s < extra_tiles)
                def _():
                    eb_ = pl.multiple_of(extra_base + s * 8, 8)
                    pltpu.sync_copy(acc.at[pl.ds(eb_, 8)],
                                    bounce.at[pl.ds(main, 8)])

            @pl.when(c == 0)
            def _():
                pltpu.sync_copy(bounce.at[pl.ds(0, main)],
                                oa.at[pl.ds(base1, main)])

            @pl.when(c == 1)
            def _():
                pltpu.sync_copy(bounce.at[pl.ds(0, main)],
                                ob.at[pl.ds(base1, main)])
            if extra_tiles:
                @pl.when((s < extra_tiles) & (c == 0))
                def _():
                    eb_ = pl.multiple_of(extra_base + s * 8, 8)
                    pltpu.sync_copy(bounce.at[pl.ds(main, 8)],
                                    oa.at[pl.ds(eb_, 8)])

                @pl.when((s < extra_tiles) & (c == 1))
                def _():
                    eb_ = pl.multiple_of(extra_base + s * 8, 8)
                    pltpu.sync_copy(bounce.at[pl.ds(main, 8)],
                                    ob.at[pl.ds(eb_, 8)])

        wb1(acc1, t1a, t1b)
        wb1(accd, tda, tdb)
        wb1(accg, dga, dgb)

    return k(eib0, eib1, ei1, eab, nfb_pad, zrows, z1d)


def _spmv(x, ei0_2d, ei1_2d, zrows):
    PER_TILE = EP // (NW * 128)
    NTRASH = 16
    K = 2

    @functools.partial(
        pl.kernel,
        out_type=(jax.ShapeDtypeStruct((N, EMB), f32),
                  jax.ShapeDtypeStruct((N, EMB), f32)),
        mesh=_mesh(),
        scratch_types=[
            pltpu.VMEM_SHARED((N + NTRASH, EMB), f32),
            pltpu.VMEM((PER_TILE, 128), i32),
            pltpu.VMEM((PER_TILE, 128), i32),
            [pltpu.VMEM((128, EMB), f32) for _ in range(K)],
            pltpu.SemaphoreType.DMA,
            pltpu.SemaphoreType.DMA,
        ],
    )
    def k(x_h, ei0_h, ei1_h, z_h, out0, out1, accum, src2, dst2, rows, gs, ss):
        s = lax.axis_index("s")
        w = _wid()
        _zero_accum(z_h, accum, s, N)
        cbase = pl.multiple_of(w * PER_TILE, 8)
        pltpu.sync_copy(ei0_h.at[pl.ds(cbase, PER_TILE)], src2)
        pltpu.sync_copy(ei1_h.at[pl.ds(cbase, PER_TILE)], dst2)
        plsc.subcore_barrier()

        def issue_gather(j, ci):
            pltpu.async_copy(x_h.at[src2.at[ci]], rows[j], gs)

        def drain_gather(j):
            pltpu.make_async_copy(x_h.at[src2.at[0]], rows[j], gs).wait()

        def issue_scatter(j, ci):
            pltpu.async_copy(rows[j], accum.at[dst2.at[ci]], ss, add=True)

        def drain_scatter(j):
            pltpu.make_async_copy(rows[j], accum.at[dst2.at[0]], ss).wait()

        for j in range(K):
            issue_gather(j, j)

        def group(g, carry):
            base = g * K
            for j in range(K):
                drain_gather(j)
            for j in range(K):
                issue_scatter(j, base + j)
            for j in range(K):
                drain_scatter(j)

            @pl.when(g < PER_TILE // K - 1)
            def _():
                for j in range(K):
                    issue_gather(j, base + K + j)
            return carry

        lax.fori_loop(0, PER_TILE // K, group, 0)
        plsc.subcore_barrier()
        _writeback(accum, out0, out1, s, N)

    return k(x, ei0_2d, ei1_2d, zrows)


def _frag_pool(pre3, a2f, batch, z128):
    C = 80
    nchunks = N // C
    per = -(-nchunks // NW)

    @functools.partial(
        pl.kernel,
        out_type=(jax.ShapeDtypeStruct((NF, EMB), f32),
                  jax.ShapeDtypeStruct((NF, EMB), f32),
                  jax.ShapeDtypeStruct((BB, EMB), f32),
                  jax.ShapeDtypeStruct((BB, EMB), f32)),
        mesh=_mesh(),
        scratch_types=[
            pltpu.VMEM_SHARED((NF, EMB), f32),
            pltpu.VMEM_SHARED((BB, EMB), f32),
            pltpu.VMEM((C,), i32),
            pltpu.VMEM((C,), i32),
            pltpu.VMEM((C, EMB), f32),
        ],
    )
    def k(x_h, a2f_h, batch_h, z_h, xf0, xf1, xap0, xap1,
          xfacc, xapacc, dstf, dstb, rows):
        s = lax.axis_index("s")
        w = _wid()
        _zero_accum(z_h, xfacc, s, NF)
        _zero_accum(z_h, xapacc, s, BB)
        plsc.subcore_barrier()

        def body(ci, carry):
            idx = w + ci * NW

            @pl.when(idx < nchunks)
            def _():
                off = pl.multiple_of(idx * C, 8)
                pltpu.sync_copy(x_h.at[pl.ds(off, C)], rows)
                pltpu.sync_copy(a2f_h.at[pl.ds(off, C)], dstf)
                pltpu.sync_copy(rows, xfacc.at[dstf], add=True)

                def relu_row(r, cc):
                    for j in range(8):
                        v = rows[r, pl.ds(j * 16, 16)]
                        rows[r, pl.ds(j * 16, 16)] = jnp.maximum(v, 0.0)
                    return cc

                lax.fori_loop(0, C, relu_row, 0)
                pltpu.sync_copy(batch_h.at[pl.ds(off, C)], dstb)
                pltpu.sync_copy(rows, xapacc.at[dstb], add=True)
            return carry

        lax.fori_loop(0, per, body, 0)
        plsc.subcore_barrier()
        _writeback(xfacc, xf0, xf1, s, NF)
        _writeback(xapacc, xap0, xap1, s, BB)

    return k(pre3, a2f, batch, z128)


def _frag_gather(xf0, xf1, fs, ft, z128):
    C = 40
    nchunks = EF // C
    per = -(-nchunks // NW)

    @functools.partial(
        pl.kernel,
        out_type=(jax.ShapeDtypeStruct((NF, EMB), f32),
                  jax.ShapeDtypeStruct((NF, EMB), f32)),
        mesh=_mesh(),
        scratch_types=[
            pltpu.VMEM_SHARED((NF, EMB), f32),
            pltpu.VMEM((C,), i32),
            pltpu.VMEM((C,), i32),
            pltpu.VMEM((C, EMB), f32),
            pltpu.VMEM((C, EMB), f32),
            pltpu.SemaphoreType.DMA,
        ],
    )
    def k(xf0_h, xf1_h, fs_h, ft_h, z_h, out0, out1,
          accum, ib, db, r0, r1, sem):
        s = lax.axis_index("s")
        w = _wid()
        _zero_accum(z_h, accum, s, NF)
        plsc.subcore_barrier()

        def body(ci, carry):
            idx = w + ci * NW

            @pl.when(idx < nchunks)
            def _():
                off = pl.multiple_of(idx * C, 8)
                pltpu.sync_copy(fs_h.at[pl.ds(off, C)], ib)
                pltpu.async_copy(xf0_h.at[ib], r0, sem).wait()
                pltpu.async_copy(xf1_h.at[ib], r1, sem).wait()

                def add_row(r, cc):
                    for j in range(8):
                        sl = pl.ds(j * 16, 16)
                        r0[r, sl] = r0[r, sl] + r1[r, sl]
                    return cc

                lax.fori_loop(0, C, add_row, 0)
                pltpu.sync_copy(ft_h.at[pl.ds(off, C)], db)
                pltpu.sync_copy(r0, accum.at[db], add=True)
            return carry

        lax.fori_loop(0, per, body, 0)
        plsc.subcore_barrier()
        _writeback(accum, out0, out1, s, NF)

    return k(xf0, xf1, fs, ft, z128)


def _layer_update(p0, p1, x, u0, u1, ue0, ue1, w, v, ve, b, relu):
    blk = 2000

    def body(p0_r, p1_r, x_r, u0_r, u1_r, ue0_r, ue1_r, w_r, v_r, ve_r, b_r,
             out_r):
        sx = p0_r[...] + p1_r[...] + x_r[...]
        acc = jnp.dot(sx, w_r[...], preferred_element_type=f32,
                      precision=lax.Precision.HIGHEST)
        acc = acc + jnp.dot(u0_r[...] + u1_r[...], v_r[...],
                            preferred_element_type=f32,
                      precision=lax.Precision.HIGHEST)
        acc = acc + jnp.dot(ue0_r[...] + ue1_r[...], ve_r[...],
                            preferred_element_type=f32,
                      precision=lax.Precision.HIGHEST)
        acc = acc + b_r[0:1, :]
        out_r[...] = jnp.maximum(acc, 0.0) if relu else acc

    return pl.pallas_call(
        body,
        grid=(N // blk,),
        in_specs=[
            pl.BlockSpec((blk, EMB), lambda i: (i, 0)),
            pl.BlockSpec((blk, EMB), lambda i: (i, 0)),
            pl.BlockSpec((blk, EMB), lambda i: (i, 0)),
            pl.BlockSpec((blk, 16), lambda i: (i, 0)),
            pl.BlockSpec((blk, 16), lambda i: (i, 0)),
            pl.BlockSpec((blk, 3), lambda i: (i, 0)),
            pl.BlockSpec((blk, 3), lambda i: (i, 0)),
            pl.BlockSpec((EMB, EMB), lambda i: (0, 0)),
            pl.BlockSpec((16, EMB), lambda i: (0, 0)),
            pl.BlockSpec((3, EMB), lambda i: (0, 0)),
            pl.BlockSpec((8, EMB), lambda i: (0, 0)),
        ],
        out_specs=pl.BlockSpec((blk, EMB), lambda i: (i, 0)),
        out_shape=jax.ShapeDtypeStruct((N, EMB), f32),
    )(p0, p1, x, u0, u1, ue0, ue1, w, v, ve, b)


def _tail(xap0, xap1, fs0, fs1, w1, b1, w2, b2, wa, wf, b3, wo, bo, oh):
    def body(xap0_r, xap1_r, fs0_r, fs1_r, w1_r, b1_r, w2_r, b2_r,
             wa_r, wf_r, b3_r, wo_r, bo_r, oh_r, out_r):
        fsum = fs0_r[...] + fs1_r[...]
        h = jnp.maximum(jnp.dot(fsum, w1_r[...], preferred_element_type=f32,
                      precision=lax.Precision.HIGHEST)
                        + b1_r[0:1, :], 0.0)
        xf = jnp.maximum(jnp.dot(h, w2_r[...], preferred_element_type=f32,
                      precision=lax.Precision.HIGHEST)
                         + b2_r[0:1, :], 0.0)
        xfp = lax.dot_general(oh_r[...], xf, (((0,), (0,)), ((), ())),
                              preferred_element_type=f32,
                              precision=lax.Precision.HIGHEST)
        xap = xap0_r[...] + xap1_r[...]
        hh = jnp.maximum(jnp.dot(xap, wa_r[...], preferred_element_type=f32,
                      precision=lax.Precision.HIGHEST)
                         + jnp.dot(xfp, wf_r[...], preferred_element_type=f32,
                      precision=lax.Precision.HIGHEST)
                         + b3_r[0:1, :], 0.0)
        out_r[...] = jnp.dot(hh, wo_r[...], preferred_element_type=f32,
                      precision=lax.Precision.HIGHEST) + bo_r[0:1, :]

    return pl.pallas_call(
        body,
        out_shape=jax.ShapeDtypeStruct((BB, EMB), f32),
    )(xap0, xap1, fs0, fs1, w1, b1, w2, b2, wa, wf, b3, wo, bo, oh)


def kernel(x_atoms, x_frags, edge_attr, node_features_bonds, edge_attr_bonds,
           params, edge_index, frag_index, batch, frag_batch, atom_to_frag_ids,
           edge_index_bonds_graph):
    ei0 = jnp.asarray(edge_index[0], i32)
    ei1 = jnp.asarray(edge_index[1], i32)
    eib0 = jnp.asarray(edge_index_bonds_graph[0], i32)
    eib1 = jnp.asarray(edge_index_bonds_graph[1], i32)
    fs = jnp.asarray(frag_index[0], i32)
    ft = jnp.asarray(frag_index[1], i32)
    a2f = jnp.asarray(atom_to_frag_ids, i32)
    bat = jnp.asarray(batch, i32)
    fbat = jnp.asarray(frag_batch, i32)
    eab = edge_attr_bonds[:, 0]

    nfb_pad = jnp.pad(node_features_bonds, ((0, 0), (0, 4)))
    x0_pad = jnp.pad(x_atoms, ((0, 0), (0, EMB - x_atoms.shape[1])))

    z625x16 = jnp.zeros((N // NS, 16), f32)
    z625x128 = jnp.zeros((N // NS, EMB), f32)
    z128x128 = jnp.zeros((128, EMB), f32)
    z1d = jnp.zeros((N // NS,), f32)

    pad_src = (jnp.arange(EP - E, dtype=i32) % N)
    pad_dst = N + (jnp.arange(EP - E, dtype=i32) % 16)
    ei0_2d = jnp.concatenate([ei0, pad_src]).reshape(EP // 128, 128)
    ei1_2d = jnp.concatenate([ei1, pad_dst]).reshape(EP // 128, 128)

    u0, u1, t1a, t1b, tda, tdb, dga, dgb = _u_build(
        eib0, eib1, ei1, eab, nfb_pad, z625x16, z1d)
    ue0 = jnp.stack([t1a, tda, dga], axis=1)
    ue1 = jnp.stack([t1b, tdb, dgb], axis=1)

    layers = params["layers"]
    x = x0_pad
    pre3 = None
    for l in range(4):
        lp = layers[l]
        wl = lp["atom_embed"]["W"]
        if l == 0:
            wl = jnp.pad(wl, ((0, EMB - wl.shape[0]), (0, 0)))
        vl = jnp.concatenate([
            lp["edge_embed"]["W"],
            jnp.zeros((4, EMB), f32),
        ], axis=0)
        ve = jnp.concatenate([
            lp["edge_attr_bond_embed"]["W"],
            (lp["edge_attr_bond_embed"]["b"] + lp["edge_embed"]["b"])[None, :],
            lp["atom_embed"]["b"][None, :],
        ], axis=0)
        bl = jnp.broadcast_to(lp["atom_embed"]["b"][None, :], (8, EMB))
        p0, p1 = _spmv(x, ei0_2d, ei1_2d, z625x128)
        out = _layer_update(p0, p1, x, u0, u1, ue0, ue1, wl, vl, ve, bl,
                            relu=(l < 3))
        if l == 3:
            pre3 = out
        else:
            x = out

    xf0, xf1, xap0, xap1 = _frag_pool(pre3, a2f, bat, z128x128)
    fs0, fs1 = _frag_gather(xf0, xf1, fs, ft, z128x128)

    lp3 = layers[3]
    w1 = lp3["frag_mlp1"]["W"]
    b1 = jnp.broadcast_to(lp3["frag_mlp1"]["b"][None, :], (8, 2 * EMB))
    w2 = lp3["frag_mlp2"]["W"]
    b2 = jnp.broadcast_to(lp3["frag_mlp2"]["b"][None, :], (8, EMB))
    wa = params["lin1"]["W"][:EMB]
    wf = params["lin1"]["W"][EMB:]
    b3 = jnp.broadcast_to(params["lin1"]["b"][None, :], (8, 2 * EMB))
    wo = jnp.pad(params["out"]["W"], ((0, 0), (0, EMB - 1)))
    bo = jnp.pad(params["out"]["b"][None, :], ((0, 7), (0, EMB - 1)))
    oh = (fbat[:, None] == jnp.arange(BB)[None, :]).astype(f32)

    res = _tail(xap0, xap1, fs0, fs1, w1, b1, w2, b2, wa, wf, b3, wo, bo, oh)
    return res[:, :1]


# speedup vs baseline: 12.0350x; 1.1889x over previous
import functools

import jax
import jax.numpy as jnp
from jax import lax
from jax.experimental import pallas as pl
from jax.experimental.pallas import tpu as pltpu
from jax.experimental.pallas import tpu_sc as plsc

N = 10000
E = 160000
NF = 2000
EF = 4000
NB = 160000
BB = 256
EMB = 128
NC = 2
NS = 16
NW = NC * NS
EP = 163840

f32 = jnp.float32
i32 = jnp.int32


def _mesh():
    return plsc.VectorSubcoreMesh(
        core_axis_name="c", subcore_axis_name="s", num_cores=NC, num_subcores=NS)


def _wid():
    return lax.axis_index("s") * NC + lax.axis_index("c")


def _row_split(total):
    main = (total // (8 * NS)) * 8
    extra_tiles = (total - main * NS) // 8
    extra_base = main * NS
    return main, extra_tiles, extra_base


def _zero_accum(z_h, accum, s, total):
    main, extra_tiles, extra_base = _row_split(total)
    base = pl.multiple_of(s * main, 8)
    pltpu.sync_copy(z_h.at[pl.ds(0, main)], accum.at[pl.ds(base, main)])
    if extra_tiles:
        @pl.when(s < extra_tiles)
        def _():
            eb = pl.multiple_of(extra_base + s * 8, 8)
            pltpu.sync_copy(z_h.at[pl.ds(0, 8)], accum.at[pl.ds(eb, 8)])


def _writeback(accum, out0, out1, s, total):
    c = lax.axis_index("c")
    main, extra_tiles, extra_base = _row_split(total)

    def _copy_ranges(out):
        base = pl.multiple_of(s * main, 8)
        pltpu.sync_copy(accum.at[pl.ds(base, main)], out.at[pl.ds(base, main)])
        if extra_tiles:
            @pl.when(s < extra_tiles)
            def _():
                eb = pl.multiple_of(extra_base + s * 8, 8)
                pltpu.sync_copy(accum.at[pl.ds(eb, 8)], out.at[pl.ds(eb, 8)])

    @pl.when(c == 0)
    def _():
        _copy_ranges(out0)

    @pl.when(c == 1)
    def _():
        _copy_ranges(out1)


def _u_build(eib0_2d, eib1_2d, eab_2d, ei1_ext, ei1p_2d, nfb_pad2, zrows, z1d):
    PER_TILE = EP // (NW * 128)
    NTRASH = 16
    onedim = jax.ShapeDtypeStruct((N,), f32)

    @functools.partial(
        pl.kernel,
        out_type=(jax.ShapeDtypeStruct((N, 16), f32),
                  jax.ShapeDtypeStruct((N, 16), f32),
                  onedim, onedim, onedim, onedim, onedim, onedim),
        mesh=_mesh(),
        scratch_types=[
            pltpu.VMEM_SHARED((N + NTRASH, 16), f32),
            pltpu.VMEM_SHARED((N + NTRASH,), f32),
            pltpu.VMEM_SHARED((N + NTRASH,), f32),
            pltpu.VMEM_SHARED((N + NTRASH,), f32),
            pltpu.VMEM((PER_TILE, 128), i32),
            pltpu.VMEM((PER_TILE, 128), i32),
            pltpu.VMEM((PER_TILE, 128), f32),
            pltpu.VMEM((PER_TILE, 128), i32),
            [pltpu.VMEM((128,), i32) for _ in range(2)],
            [pltpu.VMEM((128, 16), f32) for _ in range(2)],
            pltpu.VMEM((128,), f32),
            pltpu.VMEM((632,), f32),
            pltpu.SemaphoreType.DMA,
            pltpu.SemaphoreType.DMA,
        ],
        compiler_params=pltpu.CompilerParams(use_tc_tiling_on_sc=False),
    )
    def k(eib0_h, eib1_h, eab_h, ei1x_h, ei1p_h, nfb_h, z_h, z1_h,
          out0, out1, t1a, t1b, tda, tdb, dga, dgb,
          accum, acc1, accd, accg, b0_2, b1_2, eab_2, pb_2, dsta, rows,
          ones_b, bounce, gs, ss):
        s = lax.axis_index("s")
        w = _wid()
        _zero_accum(z_h, accum, s, N)
        main, extra_tiles, extra_base = _row_split(N)
        base1 = pl.multiple_of(s * main, 8)
        pltpu.sync_copy(z1_h.at[pl.ds(0, main)], bounce.at[pl.ds(0, main)])
        for acc in (acc1, accd, accg):
            pltpu.sync_copy(bounce.at[pl.ds(0, main)], acc.at[pl.ds(base1, main)])
        if extra_tiles:
            @pl.when(s < extra_tiles)
            def _():
                eb_ = pl.multiple_of(extra_base + s * 8, 8)
                for acc in (acc1, accd, accg):
                    pltpu.sync_copy(bounce.at[pl.ds(0, 8)], acc.at[pl.ds(eb_, 8)])
        for j in range(8):
            ones_b[pl.ds(j * 16, 16)] = jnp.ones((16,), f32)
        cbase = pl.multiple_of(w * PER_TILE, 8)
        pltpu.sync_copy(eib0_h.at[pl.ds(cbase, PER_TILE)], b0_2)
        pltpu.sync_copy(eib1_h.at[pl.ds(cbase, PER_TILE)], b1_2)
        pltpu.sync_copy(eab_h.at[pl.ds(cbase, PER_TILE)], eab_2)
        pltpu.sync_copy(ei1p_h.at[pl.ds(cbase, PER_TILE)], pb_2)
        plsc.subcore_barrier()

        def a_gather(j, ci):
            pltpu.async_copy(ei1x_h.at[b0_2.at[ci]], dsta[j], gs)
            pltpu.async_copy(nfb_h.at[b1_2.at[ci]], rows[j], gs)

        def a_drain_gather(j):
            pltpu.make_async_copy(ei1x_h.at[b0_2.at[0]], dsta[j], gs).wait()
            pltpu.make_async_copy(nfb_h.at[b1_2.at[0]], rows[j], gs).wait()

        def a_scatter(j, ci):
            pltpu.async_copy(rows[j], accum.at[dsta[j]], ss, add=True)
            pltpu.async_copy(eab_2.at[ci], acc1.at[dsta[j]], ss, add=True)
            pltpu.async_copy(ones_b, accd.at[dsta[j]], ss, add=True)

        def a_drain_scatter(j):
            pltpu.make_async_copy(rows[j], accum.at[dsta[j]], ss).wait()
            pltpu.make_async_copy(eab_2.at[0], acc1.at[dsta[j]], ss).wait()
            pltpu.make_async_copy(ones_b, accd.at[dsta[j]], ss).wait()

        for j in range(2):
            a_gather(j, j)

        def group_a(g, carry):
            base = g * 2
            for j in range(2):
                a_drain_gather(j)
            for j in range(2):
                a_scatter(j, base + j)
            for j in range(2):
                a_drain_scatter(j)

            @pl.when(g < PER_TILE // 2 - 1)
            def _():
                for j in range(2):
                    a_gather(j, base + 2 + j)
            return carry

        lax.fori_loop(0, PER_TILE // 2, group_a, 0)

        def b_gather(j, ci):
            off = pl.multiple_of((cbase + ci) * 128, 128)
            pltpu.async_copy(nfb_h.at[pl.ds(off, 128)], rows[j], gs)

        def b_drain_gather(j):
            pltpu.make_async_copy(nfb_h.at[pl.ds(0, 128)], rows[j], gs).wait()

        def b_scatter(j, ci):
            pltpu.async_copy(rows[j], accum.at[pb_2.at[ci]], ss, add=True)
            pltpu.async_copy(ones_b, accg.at[pb_2.at[ci]], ss, add=True)

        def b_drain_scatter(j, ci):
            pltpu.make_async_copy(rows[j], accum.at[pb_2.at[ci]], ss).wait()
            pltpu.make_async_copy(ones_b, accg.at[pb_2.at[ci]], ss).wait()

        for j in range(2):
            b_gather(j, j)

        def group_b(g, carry):
            base = g * 2
            for j in range(2):
                b_drain_gather(j)
            for j in range(2):
                b_scatter(j, base + j)
            for j in range(2):
                b_drain_scatter(j, base + j)

            @pl.when(g < PER_TILE // 2 - 1)
            def _():
                for j in range(2):
                    b_gather(j, base + 2 + j)
            return carry

        lax.fori_loop(0, PER_TILE // 2, group_b, 0)
        plsc.subcore_barrier()
        _writeback(accum, out0, out1, s, N)
        c = lax.axis_index("c")

        def wb1(acc, oa, ob):
            nb_ = main + (8 if extra_tiles else 0)
            pltpu.sync_copy(acc.at[pl.ds(base1, main)], bounce.at[pl.ds(0, main)])
            if extra_tiles:
                @pl.when(s < extra_tiles)
                def _():
                    eb_ = pl.multiple_of(extra_base + s * 8, 8)
                    pltpu.sync_copy(acc.at[pl.ds(eb_, 8)],
                                    bounce.at[pl.ds(main, 8)])

            @pl.when(c == 0)
            def _():
                pltpu.sync_copy(bounce.at[pl.ds(0, main)],
                                oa.at[pl.ds(base1, main)])

            @pl.when(c == 1)
            def _():
                pltpu.sync_copy(bounce.at[pl.ds(0, main)],
                                ob.at[pl.ds(base1, main)])
            if extra_tiles:
                @pl.when((s < extra_tiles) & (c == 0))
                def _():
                    eb_ = pl.multiple_of(extra_base + s * 8, 8)
                    pltpu.sync_copy(bounce.at[pl.ds(main, 8)],
                                    oa.at[pl.ds(eb_, 8)])

                @pl.when((s < extra_tiles) & (c == 1))
                def _():
                    eb_ = pl.multiple_of(extra_base + s * 8, 8)
                    pltpu.sync_copy(bounce.at[pl.ds(main, 8)],
                                    ob.at[pl.ds(eb_, 8)])

        wb1(acc1, t1a, t1b)
        wb1(accd, tda, tdb)
        wb1(accg, dga, dgb)

    return k(eib0_2d, eib1_2d, eab_2d, ei1_ext, ei1p_2d, nfb_pad2, zrows, z1d)


def _spmv(x, ei0_2d, ei1_2d, zrows):
    PER_TILE = EP // (NW * 128)
    NTRASH = 16
    K = 2

    @functools.partial(
        pl.kernel,
        out_type=(jax.ShapeDtypeStruct((N, EMB), f32),
                  jax.ShapeDtypeStruct((N, EMB), f32)),
        mesh=_mesh(),
        scratch_types=[
            pltpu.VMEM_SHARED((N + NTRASH, EMB), f32),
            pltpu.VMEM((PER_TILE, 128), i32),
            pltpu.VMEM((PER_TILE, 128), i32),
            [pltpu.VMEM((128, EMB), f32) for _ in range(K)],
            pltpu.SemaphoreType.DMA,
            pltpu.SemaphoreType.DMA,
        ],
    )
    def k(x_h, ei0_h, ei1_h, z_h, out0, out1, accum, src2, dst2, rows, gs, ss):
        s = lax.axis_index("s")
        w = _wid()
        _zero_accum(z_h, accum, s, N)
        cbase = pl.multiple_of(w * PER_TILE, 8)
        pltpu.sync_copy(ei0_h.at[pl.ds(cbase, PER_TILE)], src2)
        pltpu.sync_copy(ei1_h.at[pl.ds(cbase, PER_TILE)], dst2)
        plsc.subcore_barrier()

        def issue_gather(j, ci):
            pltpu.async_copy(x_h.at[src2.at[ci]], rows[j], gs)

        def drain_gather(j):
            pltpu.make_async_copy(x_h.at[src2.at[0]], rows[j], gs).wait()

        def issue_scatter(j, ci):
            pltpu.async_copy(rows[j], accum.at[dst2.at[ci]], ss, add=True)

        def drain_scatter(j):
            pltpu.make_async_copy(rows[j], accum.at[dst2.at[0]], ss).wait()

        for j in range(K):
            issue_gather(j, j)

        def group(g, carry):
            base = g * K
            for j in range(K):
                drain_gather(j)
            for j in range(K):
                issue_scatter(j, base + j)
            for j in range(K):
                drain_scatter(j)

            @pl.when(g < PER_TILE // K - 1)
            def _():
                for j in range(K):
                    issue_gather(j, base + K + j)
            return carry

        lax.fori_loop(0, PER_TILE // K, group, 0)
        plsc.subcore_barrier()
        _writeback(accum, out0, out1, s, N)

    return k(x, ei0_2d, ei1_2d, zrows)


def _frag_pool(pre3, a2f, batch, z128):
    C = 80
    nchunks = N // C
    per = -(-nchunks // NW)

    @functools.partial(
        pl.kernel,
        out_type=(jax.ShapeDtypeStruct((NF, EMB), f32),
                  jax.ShapeDtypeStruct((NF, EMB), f32),
                  jax.ShapeDtypeStruct((BB, EMB), f32),
                  jax.ShapeDtypeStruct((BB, EMB), f32)),
        mesh=_mesh(),
        scratch_types=[
            pltpu.VMEM_SHARED((NF, EMB), f32),
            pltpu.VMEM_SHARED((BB, EMB), f32),
            pltpu.VMEM((C,), i32),
            pltpu.VMEM((C,), i32),
            pltpu.VMEM((C, EMB), f32),
        ],
    )
    def k(x_h, a2f_h, batch_h, z_h, xf0, xf1, xap0, xap1,
          xfacc, xapacc, dstf, dstb, rows):
        s = lax.axis_index("s")
        w = _wid()
        _zero_accum(z_h, xfacc, s, NF)
        _zero_accum(z_h, xapacc, s, BB)
        plsc.subcore_barrier()

        def body(ci, carry):
            idx = w + ci * NW

            @pl.when(idx < nchunks)
            def _():
                off = pl.multiple_of(idx * C, 8)
                pltpu.sync_copy(x_h.at[pl.ds(off, C)], rows)
                pltpu.sync_copy(a2f_h.at[pl.ds(off, C)], dstf)
                pltpu.sync_copy(rows, xfacc.at[dstf], add=True)

                def relu_row(r, cc):
                    for j in range(8):
                        v = rows[r, pl.ds(j * 16, 16)]
                        rows[r, pl.ds(j * 16, 16)] = jnp.maximum(v, 0.0)
                    return cc

                lax.fori_loop(0, C, relu_row, 0)
                pltpu.sync_copy(batch_h.at[pl.ds(off, C)], dstb)
                pltpu.sync_copy(rows, xapacc.at[dstb], add=True)
            return carry

        lax.fori_loop(0, per, body, 0)
        plsc.subcore_barrier()
        _writeback(xfacc, xf0, xf1, s, NF)
        _writeback(xapacc, xap0, xap1, s, BB)

    return k(pre3, a2f, batch, z128)


def _frag_gather(xf0, xf1, fs, ft, z128):
    C = 40
    nchunks = EF // C
    per = -(-nchunks // NW)

    @functools.partial(
        pl.kernel,
        out_type=(jax.ShapeDtypeStruct((NF, EMB), f32),
                  jax.ShapeDtypeStruct((NF, EMB), f32)),
        mesh=_mesh(),
        scratch_types=[
            pltpu.VMEM_SHARED((NF, EMB), f32),
            pltpu.VMEM((C,), i32),
            pltpu.VMEM((C,), i32),
            pltpu.VMEM((C, EMB), f32),
            pltpu.VMEM((C, EMB), f32),
            pltpu.SemaphoreType.DMA,
        ],
    )
    def k(xf0_h, xf1_h, fs_h, ft_h, z_h, out0, out1,
          accum, ib, db, r0, r1, sem):
        s = lax.axis_index("s")
        w = _wid()
        _zero_accum(z_h, accum, s, NF)
        plsc.subcore_barrier()

        def body(ci, carry):
            idx = w + ci * NW

            @pl.when(idx < nchunks)
            def _():
                off = pl.multiple_of(idx * C, 8)
                pltpu.sync_copy(fs_h.at[pl.ds(off, C)], ib)
                pltpu.async_copy(xf0_h.at[ib], r0, sem).wait()
                pltpu.async_copy(xf1_h.at[ib], r1, sem).wait()

                def add_row(r, cc):
                    for j in range(8):
                        sl = pl.ds(j * 16, 16)
                        r0[r, sl] = r0[r, sl] + r1[r, sl]
                    return cc

                lax.fori_loop(0, C, add_row, 0)
                pltpu.sync_copy(ft_h.at[pl.ds(off, C)], db)
                pltpu.sync_copy(r0, accum.at[db], add=True)
            return carry

        lax.fori_loop(0, per, body, 0)
        plsc.subcore_barrier()
        _writeback(accum, out0, out1, s, NF)

    return k(xf0, xf1, fs, ft, z128)


def _layer_update(p0, p1, x, u0, u1, ue0, ue1, w, v, ve, b, relu):
    blk = 2000

    def body(p0_r, p1_r, x_r, u0_r, u1_r, ue0_r, ue1_r, w_r, v_r, ve_r, b_r,
             out_r):
        sx = p0_r[...] + p1_r[...] + x_r[...]
        acc = jnp.dot(sx, w_r[...], preferred_element_type=f32,
                      precision=lax.Precision.HIGHEST)
        acc = acc + jnp.dot(u0_r[...] + u1_r[...], v_r[...],
                            preferred_element_type=f32,
                      precision=lax.Precision.HIGHEST)
        acc = acc + jnp.dot(ue0_r[...] + ue1_r[...], ve_r[...],
                            preferred_element_type=f32,
                      precision=lax.Precision.HIGHEST)
        acc = acc + b_r[0:1, :]
        out_r[...] = jnp.maximum(acc, 0.0) if relu else acc

    return pl.pallas_call(
        body,
        grid=(N // blk,),
        in_specs=[
            pl.BlockSpec((blk, EMB), lambda i: (i, 0)),
            pl.BlockSpec((blk, EMB), lambda i: (i, 0)),
            pl.BlockSpec((blk, EMB), lambda i: (i, 0)),
            pl.BlockSpec((blk, 16), lambda i: (i, 0)),
            pl.BlockSpec((blk, 16), lambda i: (i, 0)),
            pl.BlockSpec((blk, 3), lambda i: (i, 0)),
            pl.BlockSpec((blk, 3), lambda i: (i, 0)),
            pl.BlockSpec((EMB, EMB), lambda i: (0, 0)),
            pl.BlockSpec((16, EMB), lambda i: (0, 0)),
            pl.BlockSpec((3, EMB), lambda i: (0, 0)),
            pl.BlockSpec((8, EMB), lambda i: (0, 0)),
        ],
        out_specs=pl.BlockSpec((blk, EMB), lambda i: (i, 0)),
        out_shape=jax.ShapeDtypeStruct((N, EMB), f32),
    )(p0, p1, x, u0, u1, ue0, ue1, w, v, ve, b)


def _tail(xap0, xap1, fs0, fs1, w1, b1, w2, b2, wa, wf, b3, wo, bo, oh):
    def body(xap0_r, xap1_r, fs0_r, fs1_r, w1_r, b1_r, w2_r, b2_r,
             wa_r, wf_r, b3_r, wo_r, bo_r, oh_r, out_r):
        fsum = fs0_r[...] + fs1_r[...]
        h = jnp.maximum(jnp.dot(fsum, w1_r[...], preferred_element_type=f32,
                      precision=lax.Precision.HIGHEST)
                        + b1_r[0:1, :], 0.0)
        xf = jnp.maximum(jnp.dot(h, w2_r[...], preferred_element_type=f32,
                      precision=lax.Precision.HIGHEST)
                         + b2_r[0:1, :], 0.0)
        xfp = lax.dot_general(oh_r[...], xf, (((0,), (0,)), ((), ())),
                              preferred_element_type=f32,
                              precision=lax.Precision.HIGHEST)
        xap = xap0_r[...] + xap1_r[...]
        hh = jnp.maximum(jnp.dot(xap, wa_r[...], preferred_element_type=f32,
                      precision=lax.Precision.HIGHEST)
                         + jnp.dot(xfp, wf_r[...], preferred_element_type=f32,
                      precision=lax.Precision.HIGHEST)
                         + b3_r[0:1, :], 0.0)
        out_r[...] = jnp.dot(hh, wo_r[...], preferred_element_type=f32,
                      precision=lax.Precision.HIGHEST) + bo_r[0:1, :]

    return pl.pallas_call(
        body,
        out_shape=jax.ShapeDtypeStruct((BB, EMB), f32),
    )(xap0, xap1, fs0, fs1, w1, b1, w2, b2, wa, wf, b3, wo, bo, oh)


def kernel(x_atoms, x_frags, edge_attr, node_features_bonds, edge_attr_bonds,
           params, edge_index, frag_index, batch, frag_batch, atom_to_frag_ids,
           edge_index_bonds_graph):
    ei0 = jnp.asarray(edge_index[0], i32)
    ei1 = jnp.asarray(edge_index[1], i32)
    eib0 = jnp.asarray(edge_index_bonds_graph[0], i32)
    eib1 = jnp.asarray(edge_index_bonds_graph[1], i32)
    fs = jnp.asarray(frag_index[0], i32)
    ft = jnp.asarray(frag_index[1], i32)
    a2f = jnp.asarray(atom_to_frag_ids, i32)
    bat = jnp.asarray(batch, i32)
    fbat = jnp.asarray(frag_batch, i32)
    eab = edge_attr_bonds[:, 0]

    nfb_pad2 = jnp.pad(node_features_bonds, ((0, EP - NB), (0, 4)))
    x0_pad = jnp.pad(x_atoms, ((0, 0), (0, EMB - x_atoms.shape[1])))

    z625x16 = jnp.zeros((N // NS, 16), f32)
    z625x128 = jnp.zeros((N // NS, EMB), f32)
    z128x128 = jnp.zeros((128, EMB), f32)
    z1d = jnp.zeros((N // NS,), f32)

    pad_src = (jnp.arange(EP - E, dtype=i32) % N)
    pad_dst = N + (jnp.arange(EP - E, dtype=i32) % 16)
    ei0_2d = jnp.concatenate([ei0, pad_src]).reshape(EP // 128, 128)
    ei1_2d = jnp.concatenate([ei1, pad_dst]).reshape(EP // 128, 128)

    eib0_2d = jnp.concatenate(
        [eib0, E + (jnp.arange(EP - E, dtype=i32) % 16)]).reshape(EP // 128, 128)
    eib1_2d = jnp.concatenate(
        [eib1, jnp.arange(EP - E, dtype=i32) % NB]).reshape(EP // 128, 128)
    eab_2d = jnp.concatenate(
        [eab, jnp.zeros((EP - E,), f32)]).reshape(EP // 128, 128)
    ei1_ext = jnp.concatenate([ei1, N + jnp.arange(16, dtype=i32)])

    u0, u1, t1a, t1b, tda, tdb, dga, dgb = _u_build(
        eib0_2d, eib1_2d, eab_2d, ei1_ext, ei1_2d, nfb_pad2, z625x16, z1d)
    ue0 = jnp.stack([t1a, tda, dga], axis=1)
    ue1 = jnp.stack([t1b, tdb, dgb], axis=1)

    layers = params["layers"]
    x = x0_pad
    pre3 = None
    for l in range(4):
        lp = layers[l]
        wl = lp["atom_embed"]["W"]
        if l == 0:
            wl = jnp.pad(wl, ((0, EMB - wl.shape[0]), (0, 0)))
        vl = jnp.concatenate([
            lp["edge_embed"]["W"],
            jnp.zeros((4, EMB), f32),
        ], axis=0)
        w1r = lp["edge_attr_bond_embed"]["W"]
        c2r = (lp["edge_attr_bond_embed"]["b"] + lp["edge_embed"]["b"])[None, :]
        ve = jnp.concatenate([
            w1r,
            c2r,
            1.5 * w1r + c2r + lp["atom_embed"]["b"][None, :],
        ], axis=0)
        bl = jnp.broadcast_to(lp["atom_embed"]["b"][None, :], (8, EMB))
        p0, p1 = _spmv(x, ei0_2d, ei1_2d, z625x128)
        out = _layer_update(p0, p1, x, u0, u1, ue0, ue1, wl, vl, ve, bl,
                            relu=(l < 3))
        if l == 3:
            pre3 = out
        else:
            x = out

    xf0, xf1, xap0, xap1 = _frag_pool(pre3, a2f, bat, z128x128)
    fs0, fs1 = _frag_gather(xf0, xf1, fs, ft, z128x128)

    lp3 = layers[3]
    w1 = lp3["frag_mlp1"]["W"]
    b1 = jnp.broadcast_to(lp3["frag_mlp1"]["b"][None, :], (8, 2 * EMB))
    w2 = lp3["frag_mlp2"]["W"]
    b2 = jnp.broadcast_to(lp3["frag_mlp2"]["b"][None, :], (8, EMB))
    wa = params["lin1"]["W"][:EMB]
    wf = params["lin1"]["W"][EMB:]
    b3 = jnp.broadcast_to(params["lin1"]["b"][None, :], (8, 2 * EMB))
    wo = jnp.pad(params["out"]["W"], ((0, 0), (0, EMB - 1)))
    bo = jnp.pad(params["out"]["b"][None, :], ((0, 7), (0, EMB - 1)))
    oh = (fbat[:, None] == jnp.arange(BB)[None, :]).astype(f32)

    res = _tail(xap0, xap1, fs0, fs1, w1, b1, w2, b2, wa, wf, b3, wo, bo, oh)
    return res[:, :1]


# speedup vs baseline: 13.7103x; 1.1392x over previous
import functools

import jax
import jax.numpy as jnp
from jax import lax
from jax.experimental import pallas as pl
from jax.experimental.pallas import tpu as pltpu
from jax.experimental.pallas import tpu_sc as plsc

N = 10000
E = 160000
NF = 2000
EF = 4000
NB = 160000
BB = 256
EMB = 128
NC = 2
NS = 16
NW = NC * NS
EP = 163840

f32 = jnp.float32
i32 = jnp.int32


def _mesh():
    return plsc.VectorSubcoreMesh(
        core_axis_name="c", subcore_axis_name="s", num_cores=NC, num_subcores=NS)


def _wid():
    return lax.axis_index("s") * NC + lax.axis_index("c")


def _row_split(total):
    main = (total // (8 * NS)) * 8
    extra_tiles = (total - main * NS) // 8
    extra_base = main * NS
    return main, extra_tiles, extra_base


def _zero_accum(z_h, accum, s, total):
    main, extra_tiles, extra_base = _row_split(total)
    base = pl.multiple_of(s * main, 8)
    pltpu.sync_copy(z_h.at[pl.ds(0, main)], accum.at[pl.ds(base, main)])
    if extra_tiles:
        @pl.when(s < extra_tiles)
        def _():
            eb = pl.multiple_of(extra_base + s * 8, 8)
            pltpu.sync_copy(z_h.at[pl.ds(0, 8)], accum.at[pl.ds(eb, 8)])


def _writeback(accum, out0, out1, s, total):
    c = lax.axis_index("c")
    main, extra_tiles, extra_base = _row_split(total)

    def _copy_ranges(out):
        base = pl.multiple_of(s * main, 8)
        pltpu.sync_copy(accum.at[pl.ds(base, main)], out.at[pl.ds(base, main)])
        if extra_tiles:
            @pl.when(s < extra_tiles)
            def _():
                eb = pl.multiple_of(extra_base + s * 8, 8)
                pltpu.sync_copy(accum.at[pl.ds(eb, 8)], out.at[pl.ds(eb, 8)])

    @pl.when(c == 0)
    def _():
        _copy_ranges(out0)

    @pl.when(c == 1)
    def _():
        _copy_ranges(out1)


def _u_build(eib0_2d, eib1_2d, eab_2d, ei1_ext, ei1p_2d, nfb_pad2, zrows, z1d):
    PER_TILE = EP // (NW * 128)
    NTRASH = 16
    onedim = jax.ShapeDtypeStruct((N,), f32)

    @functools.partial(
        pl.kernel,
        out_type=(jax.ShapeDtypeStruct((N, 16), f32),
                  jax.ShapeDtypeStruct((N, 16), f32),
                  onedim, onedim, onedim, onedim, onedim, onedim),
        mesh=_mesh(),
        scratch_types=[
            pltpu.VMEM_SHARED((N + NTRASH, 16), f32),
            pltpu.VMEM_SHARED((N + NTRASH,), f32),
            pltpu.VMEM_SHARED((N + NTRASH,), f32),
            pltpu.VMEM_SHARED((N + NTRASH,), f32),
            pltpu.VMEM((PER_TILE, 128), i32),
            pltpu.VMEM((PER_TILE, 128), i32),
            pltpu.VMEM((PER_TILE, 128), f32),
            pltpu.VMEM((PER_TILE, 128), i32),
            [pltpu.VMEM((128,), i32) for _ in range(2)],
            [pltpu.VMEM((128, 16), f32) for _ in range(2)],
            pltpu.VMEM((128,), f32),
            pltpu.VMEM((632,), f32),
            pltpu.SemaphoreType.DMA,
            pltpu.SemaphoreType.DMA,
        ],
        compiler_params=pltpu.CompilerParams(use_tc_tiling_on_sc=False),
    )
    def k(eib0_h, eib1_h, eab_h, ei1x_h, ei1p_h, nfb_h, z_h, z1_h,
          out0, out1, t1a, t1b, tda, tdb, dga, dgb,
          accum, acc1, accd, accg, b0_2, b1_2, eab_2, pb_2, dsta, rows,
          ones_b, bounce, gs, ss):
        s = lax.axis_index("s")
        w = _wid()
        _zero_accum(z_h, accum, s, N)
        main, extra_tiles, extra_base = _row_split(N)
        base1 = pl.multiple_of(s * main, 8)
        pltpu.sync_copy(z1_h.at[pl.ds(0, main)], bounce.at[pl.ds(0, main)])
        for acc in (acc1, accd, accg):
            pltpu.sync_copy(bounce.at[pl.ds(0, main)], acc.at[pl.ds(base1, main)])
        if extra_tiles:
            @pl.when(s < extra_tiles)
            def _():
                eb_ = pl.multiple_of(extra_base + s * 8, 8)
                for acc in (acc1, accd, accg):
                    pltpu.sync_copy(bounce.at[pl.ds(0, 8)], acc.at[pl.ds(eb_, 8)])
        for j in range(8):
            ones_b[pl.ds(j * 16, 16)] = jnp.ones((16,), f32)
        cbase = pl.multiple_of(w * PER_TILE, 8)
        pltpu.sync_copy(eib0_h.at[pl.ds(cbase, PER_TILE)], b0_2)
        pltpu.sync_copy(eib1_h.at[pl.ds(cbase, PER_TILE)], b1_2)
        pltpu.sync_copy(eab_h.at[pl.ds(cbase, PER_TILE)], eab_2)
        pltpu.sync_copy(ei1p_h.at[pl.ds(cbase, PER_TILE)], pb_2)
        plsc.subcore_barrier()

        def a_gather(j, ci):
            pltpu.async_copy(ei1x_h.at[b0_2.at[ci]], dsta[j], gs)
            pltpu.async_copy(nfb_h.at[b1_2.at[ci]], rows[j], gs)

        def a_drain_gather(j):
            pltpu.make_async_copy(ei1x_h.at[b0_2.at[0]], dsta[j], gs).wait()
            pltpu.make_async_copy(nfb_h.at[b1_2.at[0]], rows[j], gs).wait()

        def a_scatter(j, ci):
            pltpu.async_copy(rows[j], accum.at[dsta[j]], ss, add=True)
            pltpu.async_copy(eab_2.at[ci], acc1.at[dsta[j]], ss, add=True)
            pltpu.async_copy(ones_b, accd.at[dsta[j]], ss, add=True)

        def a_drain_scatter(j):
            pltpu.make_async_copy(rows[j], accum.at[dsta[j]], ss).wait()
            pltpu.make_async_copy(eab_2.at[0], acc1.at[dsta[j]], ss).wait()
            pltpu.make_async_copy(ones_b, accd.at[dsta[j]], ss).wait()

        for j in range(2):
            a_gather(j, j)

        def group_a(g, carry):
            base = g * 2
            for j in range(2):
                a_drain_gather(j)
            for j in range(2):
                a_scatter(j, base + j)
            for j in range(2):
                a_drain_scatter(j)

            @pl.when(g < PER_TILE // 2 - 1)
            def _():
                for j in range(2):
                    a_gather(j, base + 2 + j)
            return carry

        lax.fori_loop(0, PER_TILE // 2, group_a, 0)

        def b_gather(j, ci):
            off = pl.multiple_of((cbase + ci) * 128, 128)
            pltpu.async_copy(nfb_h.at[pl.ds(off, 128)], rows[j], gs)

        def b_drain_gather(j):
            pltpu.make_async_copy(nfb_h.at[pl.ds(0, 128)], rows[j], gs).wait()

        def b_scatter(j, ci):
            pltpu.async_copy(rows[j], accum.at[pb_2.at[ci]], ss, add=True)
            pltpu.async_copy(ones_b, accg.at[pb_2.at[ci]], ss, add=True)

        def b_drain_scatter(j, ci):
            pltpu.make_async_copy(rows[j], accum.at[pb_2.at[ci]], ss).wait()
            pltpu.make_async_copy(ones_b, accg.at[pb_2.at[ci]], ss).wait()

        for j in range(2):
            b_gather(j, j)

        def group_b(g, carry):
            base = g * 2
            for j in range(2):
                b_drain_gather(j)
            for j in range(2):
                b_scatter(j, base + j)
            for j in range(2):
                b_drain_scatter(j, base + j)

            @pl.when(g < PER_TILE // 2 - 1)
            def _():
                for j in range(2):
                    b_gather(j, base + 2 + j)
            return carry

        lax.fori_loop(0, PER_TILE // 2, group_b, 0)
        plsc.subcore_barrier()
        _writeback(accum, out0, out1, s, N)
        c = lax.axis_index("c")

        def wb1(acc, oa, ob):
            nb_ = main + (8 if extra_tiles else 0)
            pltpu.sync_copy(acc.at[pl.ds(base1, main)], bounce.at[pl.ds(0, main)])
            if extra_tiles:
                @pl.when(s < extra_tiles)
                def _():
                    eb_ = pl.multiple_of(extra_base + s * 8, 8)
                    pltpu.sync_copy(acc.at[pl.ds(eb_, 8)],
                                    bounce.at[pl.ds(main, 8)])

            @pl.when(c == 0)
            def _():
                pltpu.sync_copy(bounce.at[pl.ds(0, main)],
                                oa.at[pl.ds(base1, main)])

            @pl.when(c == 1)
            def _():
                pltpu.sync_copy(bounce.at[pl.ds(0, main)],
                                ob.at[pl.ds(base1, main)])
            if extra_tiles:
                @pl.when((s < extra_tiles) & (c == 0))
                def _():
                    eb_ = pl.multiple_of(extra_base + s * 8, 8)
                    pltpu.sync_copy(bounce.at[pl.ds(main, 8)],
                                    oa.at[pl.ds(eb_, 8)])

                @pl.when((s < extra_tiles) & (c == 1))
                def _():
                    eb_ = pl.multiple_of(extra_base + s * 8, 8)
                    pltpu.sync_copy(bounce.at[pl.ds(main, 8)],
                                    ob.at[pl.ds(eb_, 8)])

        wb1(acc1, t1a, t1b)
        wb1(accd, tda, tdb)
        wb1(accg, dga, dgb)

    return k(eib0_2d, eib1_2d, eab_2d, ei1_ext, ei1p_2d, nfb_pad2, zrows, z1d)


def _spmv(x, ei0_2d, ei1_2d, zrows):
    C = 128
    PER_TILE = EP // (NW * C)
    NTRASH = 16
    NSTEP = 42

    @functools.partial(
        pl.kernel,
        out_type=(jax.ShapeDtypeStruct((N, EMB), f32),
                  jax.ShapeDtypeStruct((N, EMB), f32)),
        mesh=_mesh(),
        scratch_types=[
            pltpu.VMEM_SHARED((N + NTRASH, EMB), f32),
            pltpu.VMEM((PER_TILE, C), i32),
            pltpu.VMEM((PER_TILE, C), i32),
            [pltpu.VMEM((C, EMB), f32) for _ in range(2)],
            [pltpu.SemaphoreType.DMA for _ in range(2)],
            [pltpu.SemaphoreType.DMA for _ in range(2)],
        ],
    )
    def k(x_h, ei0_h, ei1_h, z_h, out0, out1, accum, src2, dst2, rows, gs, ss):
        s = lax.axis_index("s")
        w = _wid()
        _zero_accum(z_h, accum, s, N)
        cbase = pl.multiple_of(w * PER_TILE, 8)
        pltpu.sync_copy(ei0_h.at[pl.ds(cbase, PER_TILE)], src2)
        pltpu.sync_copy(ei1_h.at[pl.ds(cbase, PER_TILE)], dst2)
        plsc.subcore_barrier()

        def issue_g(p, ci):
            pltpu.async_copy(x_h.at[src2.at[ci]], rows[p], gs[p])

        def drain_g(p):
            pltpu.make_async_copy(x_h.at[src2.at[0]], rows[p], gs[p]).wait()

        def issue_s(p, ci):
            pltpu.async_copy(rows[p], accum.at[dst2.at[ci]], ss[p], add=True)

        def drain_s(p):
            pltpu.make_async_copy(rows[p], accum.at[dst2.at[0]], ss[p]).wait()

        issue_g(0, 0)
        issue_g(1, 1)

        def body(g, carry):
            for p in range(2):
                ci = g * 2 + p
                drain_g(p)
                issue_s(p, ci)
                drain_s(p)

                @pl.when(ci + 2 < PER_TILE)
                def _():
                    issue_g(p, ci + 2)
            return carry

        lax.fori_loop(0, PER_TILE // 2, body, 0)
        plsc.subcore_barrier()
        _writeback(accum, out0, out1, s, N)

    return k(x, ei0_2d, ei1_2d, zrows)


def _frag_pool(pre3, a2f, batch, z128):
    C = 80
    nchunks = N // C
    per = -(-nchunks // NW)

    @functools.partial(
        pl.kernel,
        out_type=(jax.ShapeDtypeStruct((NF, EMB), f32),
                  jax.ShapeDtypeStruct((NF, EMB), f32),
                  jax.ShapeDtypeStruct((BB, EMB), f32),
                  jax.ShapeDtypeStruct((BB, EMB), f32)),
        mesh=_mesh(),
        scratch_types=[
            pltpu.VMEM_SHARED((NF, EMB), f32),
            pltpu.VMEM_SHARED((BB, EMB), f32),
            pltpu.VMEM((C,), i32),
            pltpu.VMEM((C,), i32),
            pltpu.VMEM((C, EMB), f32),
        ],
    )
    def k(x_h, a2f_h, batch_h, z_h, xf0, xf1, xap0, xap1,
          xfacc, xapacc, dstf, dstb, rows):
        s = lax.axis_index("s")
        w = _wid()
        _zero_accum(z_h, xfacc, s, NF)
        _zero_accum(z_h, xapacc, s, BB)
        plsc.subcore_barrier()

        def body(ci, carry):
            idx = w + ci * NW

            @pl.when(idx < nchunks)
            def _():
                off = pl.multiple_of(idx * C, 8)
                pltpu.sync_copy(x_h.at[pl.ds(off, C)], rows)
                pltpu.sync_copy(a2f_h.at[pl.ds(off, C)], dstf)
                pltpu.sync_copy(rows, xfacc.at[dstf], add=True)

                def relu_row(r, cc):
                    for j in range(8):
                        v = rows[r, pl.ds(j * 16, 16)]
                        rows[r, pl.ds(j * 16, 16)] = jnp.maximum(v, 0.0)
                    return cc

                lax.fori_loop(0, C, relu_row, 0)
                pltpu.sync_copy(batch_h.at[pl.ds(off, C)], dstb)
                pltpu.sync_copy(rows, xapacc.at[dstb], add=True)
            return carry

        lax.fori_loop(0, per, body, 0)
        plsc.subcore_barrier()
        _writeback(xfacc, xf0, xf1, s, NF)
        _writeback(xapacc, xap0, xap1, s, BB)

    return k(pre3, a2f, batch, z128)


def _frag_gather(xf0, xf1, fs, ft, z128):
    C = 40
    nchunks = EF // C
    per = -(-nchunks // NW)

    @functools.partial(
        pl.kernel,
        out_type=(jax.ShapeDtypeStruct((NF, EMB), f32),
                  jax.ShapeDtypeStruct((NF, EMB), f32)),
        mesh=_mesh(),
        scratch_types=[
            pltpu.VMEM_SHARED((NF, EMB), f32),
            pltpu.VMEM((C,), i32),
            pltpu.VMEM((C,), i32),
            pltpu.VMEM((C, EMB), f32),
            pltpu.VMEM((C, EMB), f32),
            pltpu.SemaphoreType.DMA,
        ],
    )
    def k(xf0_h, xf1_h, fs_h, ft_h, z_h, out0, out1,
          accum, ib, db, r0, r1, sem):
        s = lax.axis_index("s")
        w = _wid()
        _zero_accum(z_h, accum, s, NF)
        plsc.subcore_barrier()

        def body(ci, carry):
            idx = w + ci * NW

            @pl.when(idx < nchunks)
            def _():
                off = pl.multiple_of(idx * C, 8)
                pltpu.sync_copy(fs_h.at[pl.ds(off, C)], ib)
                pltpu.async_copy(xf0_h.at[ib], r0, sem).wait()
                pltpu.async_copy(xf1_h.at[ib], r1, sem).wait()

                def add_row(r, cc):
                    for j in range(8):
                        sl = pl.ds(j * 16, 16)
                        r0[r, sl] = r0[r, sl] + r1[r, sl]
                    return cc

                lax.fori_loop(0, C, add_row, 0)
                pltpu.sync_copy(ft_h.at[pl.ds(off, C)], db)
                pltpu.sync_copy(r0, accum.at[db], add=True)
            return carry

        lax.fori_loop(0, per, body, 0)
        plsc.subcore_barrier()
        _writeback(accum, out0, out1, s, NF)

    return k(xf0, xf1, fs, ft, z128)


def _layer_update(p0, p1, x, u0, u1, ue0, ue1, w, v, ve, b, relu):
    blk = 2000

    def body(p0_r, p1_r, x_r, u0_r, u1_r, ue0_r, ue1_r, w_r, v_r, ve_r, b_r,
             out_r):
        sx = p0_r[...] + p1_r[...] + x_r[...]
        acc = jnp.dot(sx, w_r[...], preferred_element_type=f32,
                      precision=lax.Precision.HIGHEST)
        acc = acc + jnp.dot(u0_r[...] + u1_r[...], v_r[...],
                            preferred_element_type=f32,
                      precision=lax.Precision.HIGHEST)
        acc = acc + jnp.dot(ue0_r[...] + ue1_r[...], ve_r[...],
                            preferred_element_type=f32,
                      precision=lax.Precision.HIGHEST)
        acc = acc + b_r[0:1, :]
        out_r[...] = jnp.maximum(acc, 0.0) if relu else acc

    return pl.pallas_call(
        body,
        grid=(N // blk,),
        in_specs=[
            pl.BlockSpec((blk, EMB), lambda i: (i, 0)),
            pl.BlockSpec((blk, EMB), lambda i: (i, 0)),
            pl.BlockSpec((blk, EMB), lambda i: (i, 0)),
            pl.BlockSpec((blk, 16), lambda i: (i, 0)),
            pl.BlockSpec((blk, 16), lambda i: (i, 0)),
            pl.BlockSpec((blk, 3), lambda i: (i, 0)),
            pl.BlockSpec((blk, 3), lambda i: (i, 0)),
            pl.BlockSpec((EMB, EMB), lambda i: (0, 0)),
            pl.BlockSpec((16, EMB), lambda i: (0, 0)),
            pl.BlockSpec((3, EMB), lambda i: (0, 0)),
            pl.BlockSpec((8, EMB), lambda i: (0, 0)),
        ],
        out_specs=pl.BlockSpec((blk, EMB), lambda i: (i, 0)),
        out_shape=jax.ShapeDtypeStruct((N, EMB), f32),
    )(p0, p1, x, u0, u1, ue0, ue1, w, v, ve, b)


def _tail(xap0, xap1, fs0, fs1, w1, b1, w2, b2, wa, wf, b3, wo, bo, oh):
    def body(xap0_r, xap1_r, fs0_r, fs1_r, w1_r, b1_r, w2_r, b2_r,
             wa_r, wf_r, b3_r, wo_r, bo_r, oh_r, out_r):
        fsum = fs0_r[...] + fs1_r[...]
        h = jnp.maximum(jnp.dot(fsum, w1_r[...], preferred_element_type=f32,
                      precision=lax.Precision.HIGHEST)
                        + b1_r[0:1, :], 0.0)
        xf = jnp.maximum(jnp.dot(h, w2_r[...], preferred_element_type=f32,
                      precision=lax.Precision.HIGHEST)
                         + b2_r[0:1, :], 0.0)
        xfp = lax.dot_general(oh_r[...], xf, (((0,), (0,)), ((), ())),
                              preferred_element_type=f32,
                              precision=lax.Precision.HIGHEST)
        xap = xap0_r[...] + xap1_r[...]
        hh = jnp.maximum(jnp.dot(xap, wa_r[...], preferred_element_type=f32,
                      precision=lax.Precision.HIGHEST)
                         + jnp.dot(xfp, wf_r[...], preferred_element_type=f32,
                      precision=lax.Precision.HIGHEST)
                         + b3_r[0:1, :], 0.0)
        out_r[...] = jnp.dot(hh, wo_r[...], preferred_element_type=f32,
                      precision=lax.Precision.HIGHEST) + bo_r[0:1, :]

    return pl.pallas_call(
        body,
        out_shape=jax.ShapeDtypeStruct((BB, EMB), f32),
    )(xap0, xap1, fs0, fs1, w1, b1, w2, b2, wa, wf, b3, wo, bo, oh)


def kernel(x_atoms, x_frags, edge_attr, node_features_bonds, edge_attr_bonds,
           params, edge_index, frag_index, batch, frag_batch, atom_to_frag_ids,
           edge_index_bonds_graph):
    ei0 = jnp.asarray(edge_index[0], i32)
    ei1 = jnp.asarray(edge_index[1], i32)
    eib0 = jnp.asarray(edge_index_bonds_graph[0], i32)
    eib1 = jnp.asarray(edge_index_bonds_graph[1], i32)
    fs = jnp.asarray(frag_index[0], i32)
    ft = jnp.asarray(frag_index[1], i32)
    a2f = jnp.asarray(atom_to_frag_ids, i32)
    bat = jnp.asarray(batch, i32)
    fbat = jnp.asarray(frag_batch, i32)
    eab = edge_attr_bonds[:, 0]

    nfb_pad2 = jnp.pad(node_features_bonds, ((0, EP - NB), (0, 4)))
    x0_pad = jnp.pad(x_atoms, ((0, 0), (0, EMB - x_atoms.shape[1])))

    z625x16 = jnp.zeros((N // NS, 16), f32)
    z625x128 = jnp.zeros((N // NS, EMB), f32)
    z128x128 = jnp.zeros((128, EMB), f32)
    z1d = jnp.zeros((N // NS,), f32)

    pad_src = (jnp.arange(EP - E, dtype=i32) % N)
    pad_dst = N + (jnp.arange(EP - E, dtype=i32) % 16)
    ei1_pad = jnp.concatenate([ei1, pad_dst])
    ei0_2d = jnp.concatenate([ei0, pad_src]).reshape(EP // 128, 128)
    ei1_2d = ei1_pad.reshape(EP // 128, 128)
    ei1p128 = ei1_2d

    eib0_2d = jnp.concatenate(
        [eib0, E + (jnp.arange(EP - E, dtype=i32) % 16)]).reshape(EP // 128, 128)
    eib1_2d = jnp.concatenate(
        [eib1, jnp.arange(EP - E, dtype=i32) % NB]).reshape(EP // 128, 128)
    eab_2d = jnp.concatenate(
        [eab, jnp.zeros((EP - E,), f32)]).reshape(EP // 128, 128)
    ei1_ext = jnp.concatenate([ei1, N + jnp.arange(16, dtype=i32)])

    u0, u1, t1a, t1b, tda, tdb, dga, dgb = _u_build(
        eib0_2d, eib1_2d, eab_2d, ei1_ext, ei1p128, nfb_pad2, z625x16, z1d)
    ue0 = jnp.stack([t1a, tda, dga], axis=1)
    ue1 = jnp.stack([t1b, tdb, dgb], axis=1)

    layers = params["layers"]
    x = x0_pad
    pre3 = None
    for l in range(4):
        lp = layers[l]
        wl = lp["atom_embed"]["W"]
        if l == 0:
            wl = jnp.pad(wl, ((0, EMB - wl.shape[0]), (0, 0)))
        vl = jnp.concatenate([
            lp["edge_embed"]["W"],
            jnp.zeros((4, EMB), f32),
        ], axis=0)
        w1r = lp["edge_attr_bond_embed"]["W"]
        c2r = (lp["edge_attr_bond_embed"]["b"] + lp["edge_embed"]["b"])[None, :]
        ve = jnp.concatenate([
            w1r,
            c2r,
            1.5 * w1r + c2r + lp["atom_embed"]["b"][None, :],
        ], axis=0)
        bl = jnp.broadcast_to(lp["atom_embed"]["b"][None, :], (8, EMB))
        p0, p1 = _spmv(x, ei0_2d, ei1_2d, z625x128)
        out = _layer_update(p0, p1, x, u0, u1, ue0, ue1, wl, vl, ve, bl,
                            relu=(l < 3))
        if l == 3:
            pre3 = out
        else:
            x = out

    xf0, xf1, xap0, xap1 = _frag_pool(pre3, a2f, bat, z128x128)
    fs0, fs1 = _frag_gather(xf0, xf1, fs, ft, z128x128)

    lp3 = layers[3]
    w1 = lp3["frag_mlp1"]["W"]
    b1 = jnp.broadcast_to(lp3["frag_mlp1"]["b"][None, :], (8, 2 * EMB))
    w2 = lp3["frag_mlp2"]["W"]
    b2 = jnp.broadcast_to(lp3["frag_mlp2"]["b"][None, :], (8, EMB))
    wa = params["lin1"]["W"][:EMB]
    wf = params["lin1"]["W"][EMB:]
    b3 = jnp.broadcast_to(params["lin1"]["b"][None, :], (8, 2 * EMB))
    wo = jnp.pad(params["out"]["W"], ((0, 0), (0, EMB - 1)))
    bo = jnp.pad(params["out"]["b"][None, :], ((0, 7), (0, EMB - 1)))
    oh = (fbat[:, None] == jnp.arange(BB)[None, :]).astype(f32)

    res = _tail(xap0, xap1, fs0, fs1, w1, b1, w2, b2, wa, wf, b3, wo, bo, oh)
    return res[:, :1]
